# baseline (speedup 1.0000x reference)
import functools

import jax
import jax.numpy as jnp
import numpy as np
from jax import lax
from jax.experimental import pallas as pl
from jax.experimental.pallas import tpu as pltpu
from jax.experimental.pallas import tpu_sc as plsc

_ANCHORS = np.array([[[10.0, 13.0], [16.0, 30.0], [33.0, 23.0]],
                     [[30.0, 61.0], [62.0, 45.0], [59.0, 119.0]],
                     [[116.0, 90.0], [156.0, 198.0], [373.0, 326.0]]],
                    dtype=np.float32)
_ANCHOR_T = 0.2
_L_XY, _L_WH, _L_OBJ, _L_CLS = 2.0, 2.0, 1.0, 0.5

_NT = 1024
_NW = 32
_TPW = _NT // _NW
_CH = 240
_F = 122880
_RTC = 1920


def _softplus(x):
    return jnp.maximum(x, 0.0) + jnp.log1p(jnp.exp(-jnp.abs(x)))


def _sc_body(grids, q0_hbm, q1_hbm, q2_hbm, t0_hbm, t1_hbm, t2_hbm, t3_hbm,
             t4_hbm, t5_hbm, sc_hbm,
             o0_hbm, o1_hbm, o2_hbm, p0_hbm, p1_hbm, p2_hbm,
             sbufA, sbufB, cbuf, tb0, tb2, tb3, tb4, tb5, scv, idxb, rowsb,
             semI, semO, semG):
    wid = lax.axis_index("s") * 2 + lax.axis_index("c")
    iota16 = lax.iota(jnp.int32, 16)
    col4 = jnp.full((16,), 4, jnp.int32)

    tbase = wid * _TPW
    pltpu.sync_copy(t0_hbm.at[pl.ds(tbase, _TPW)], tb0)
    pltpu.sync_copy(t2_hbm.at[pl.ds(tbase, _TPW)], tb2)
    pltpu.sync_copy(t3_hbm.at[pl.ds(tbase, _TPW)], tb3)
    pltpu.sync_copy(t4_hbm.at[pl.ds(tbase, _TPW)], tb4)
    pltpu.sync_copy(t5_hbm.at[pl.ds(tbase, _TPW)], tb5)
    pltpu.sync_copy(sc_hbm, scv)

    gdescs = []
    for level, (qh, (ny, nx)) in enumerate(
            ((q0_hbm, grids[0]), (q1_hbm, grids[1]), (q2_hbm, grids[2]))):
        for j in range(_TPW // 16):
            sl = pl.ds(j * 16, 16)
            s = scv[pl.ds(level * 16, 16)]
            tw = tb4[sl]
            th = tb5[sl]
            best = None
            a = None
            for k in range(3):
                aw = float(_ANCHORS[level, k, 0])
                ah = float(_ANCHORS[level, k, 1])
                inter = jnp.minimum(aw, tw) * jnp.minimum(ah, th)
                iou = inter / (aw * ah + tw * th - inter)
                if k == 0:
                    best = iou
                    a = jnp.zeros((16,), jnp.int32)
                else:
                    a = jnp.where(iou > best, k, a)
                    best = jnp.maximum(iou, best)
            b = tb0[sl].astype(jnp.int32)
            gi = (tb2[sl] * s).astype(jnp.int32)
            gj = (tb3[sl] * s).astype(jnp.int32)
            if level == 2:
                lin = ((a * ny + gj) * nx + gi) * 16 + b
            else:
                lin = ((b * 3 + a) * ny + gj) * nx + gi
            idxb[sl] = lin
        for j2 in range(_TPW // 16):
            linv = idxb[pl.ds(j2 * 16, 16)]
            for jj in range(16):
                j = j2 * 16 + jj
                r = linv[jj]
                gdescs.append(pltpu.async_copy(
                    qh.at[pl.ds(r, 1), :],
                    rowsb.at[pl.ds(level * _TPW + j, 1), :], semG))

    def extract(sbuf, cslot, nrows, clamp, row0, oh):
        ngroups = (nrows + 15) // 16

        def g_body(g, carry):
            rows = g * 16 + iota16
            if clamp:
                rows = jnp.minimum(rows, nrows - 1)
            vals = plsc.load_gather(sbuf, [rows, col4])
            cbuf[pl.ds(cslot * _CH + g * 16, 16)] = vals
            return carry

        lax.fori_loop(0, ngroups, g_body, 0)
        return pltpu.async_copy(cbuf.at[pl.ds(cslot * _CH, nrows)],
                                oh.at[pl.ds(row0, nrows)], semO)

    def stream_level(qh, oh, rows_pt, qoff):
        base = wid * rows_pt
        npairs = rows_pt // (2 * _CH)
        rem = rows_pt - 2 * npairs * _CH

        def in_copy(c, sbuf):
            return pltpu.async_copy(
                qh.at[pl.ds(qoff + base + c * _CH, _CH), :], sbuf, semI)

        in_copy(0, sbufA)

        def pair(pr, carry):
            c0 = 2 * pr
            in_copy(c0 + 1, sbufB)
            pltpu.make_async_copy(qh.at[pl.ds(qoff, _CH), :], sbufA,
                                  semI).wait()
            dA = extract(sbufA, 0, _CH, False, base + c0 * _CH, oh)

            @pl.when(pr < npairs - 1)
            def _():
                in_copy(c0 + 2, sbufA)

            pltpu.make_async_copy(qh.at[pl.ds(qoff, _CH), :], sbufB,
                                  semI).wait()
            dB = extract(sbufB, 1, _CH, False, base + (c0 + 1) * _CH, oh)
            dA.wait()
            dB.wait()
            return carry

        lax.fori_loop(0, npairs, pair, 0)
        if rem:
            row0 = base + 2 * npairs * _CH
            pltpu.sync_copy(qh.at[pl.ds(qoff + row0, rem), :],
                            sbufA.at[pl.ds(0, rem), :])
            extract(sbufA, 0, rem, True, row0, oh).wait()

    stream_level(q0_hbm, o0_hbm, (q0_hbm.shape[0] - _F) // _NW, _F)
    stream_level(q1_hbm, o1_hbm, q1_hbm.shape[0] // _NW, 0)
    stream_level(q2_hbm, o2_hbm, q2_hbm.shape[0] // _NW, 0)

    for d in gdescs:
        d.wait()
    for level, ph in enumerate((p0_hbm, p1_hbm, p2_hbm)):
        pltpu.sync_copy(rowsb.at[pl.ds(level * _TPW, _TPW), :],
                        ph.at[pl.ds(tbase, _TPW), :])


def _sc_stage(q0, q1, q2, tcols, scales):
    grids = ((80, 80), (40, 40), (20, 20))
    n0, n1, n2 = q0.shape[0], q1.shape[0], q2.shape[0]
    mesh = plsc.VectorSubcoreMesh(core_axis_name="c", subcore_axis_name="s")
    kfn = functools.partial(
        pl.kernel,
        mesh=mesh,
        out_type=(jax.ShapeDtypeStruct((n0 - _F,), jnp.float32),
                  jax.ShapeDtypeStruct((n1,), jnp.float32),
                  jax.ShapeDtypeStruct((n2,), jnp.float32),
                  jax.ShapeDtypeStruct((_NT, 85), jnp.float32),
                  jax.ShapeDtypeStruct((_NT, 85), jnp.float32),
                  jax.ShapeDtypeStruct((_NT, 85), jnp.float32)),
        scratch_types=[
            pltpu.VMEM((_CH, 85), jnp.float32),
            pltpu.VMEM((_CH, 85), jnp.float32),
            pltpu.VMEM((2 * _CH,), jnp.float32),
            pltpu.VMEM((_TPW,), jnp.float32),
            pltpu.VMEM((_TPW,), jnp.float32),
            pltpu.VMEM((_TPW,), jnp.float32),
            pltpu.VMEM((_TPW,), jnp.float32),
            pltpu.VMEM((_TPW,), jnp.float32),
            pltpu.VMEM((48,), jnp.float32),
            pltpu.VMEM((_TPW,), jnp.int32),
            pltpu.VMEM((3 * _TPW, 85), jnp.float32),
            pltpu.SemaphoreType.DMA,
            pltpu.SemaphoreType.DMA,
            pltpu.SemaphoreType.DMA,
        ],
        compiler_params=pltpu.CompilerParams(needs_layout_passes=False),
    )(functools.partial(_sc_body, grids))
    return kfn(q0, q1, q2, *tcols, scales)


def _tc_stream_body(q_ref, out_ref):
    i = pl.program_id(0)

    @pl.when(i == 0)
    def _():
        out_ref[...] = jnp.zeros_like(out_ref)

    lane = jax.lax.broadcasted_iota(jnp.int32, (_RTC, 85), 1)
    x = jnp.where(lane == 4, q_ref[...], -1e9)
    out_ref[...] += jnp.full((1, 1), jnp.sum(jnp.log1p(jnp.exp(x))),
                             jnp.float32)


def _tc_stream_call(q0):
    return pl.pallas_call(
        _tc_stream_body,
        grid=(_F // _RTC,),
        in_specs=[pl.BlockSpec((_RTC, 85), lambda i: (i, 0))],
        out_specs=pl.BlockSpec((1, 1), lambda i: (0, 0)),
        out_shape=jax.ShapeDtypeStruct((1, 1), jnp.float32),
        compiler_params=pltpu.CompilerParams(
            dimension_semantics=("arbitrary",)),
    )(q0)


def _assign_cols(t_ref, img, level, ny, nx):
    s = jnp.float32(ny) / img
    tw = t_ref[:, 4:5] * s
    th = t_ref[:, 5:6] * s
    ious = []
    for k in range(3):
        aw = _ANCHORS[level, k, 0] * s
        ah = _ANCHORS[level, k, 1] * s
        inter = jnp.minimum(aw, tw) * jnp.minimum(ah, th)
        ious.append(inter / (aw * ah + tw * th - inter))
    a = jnp.where(ious[1] > ious[0], 1, 0)
    best = jnp.maximum(ious[0], ious[1])
    a = jnp.where(ious[2] > best, 2, a)
    best = jnp.maximum(best, ious[2])
    valid = best > _ANCHOR_T
    b = t_ref[:, 0:1].astype(jnp.int32)
    gx = t_ref[:, 2:3] * s
    gy = t_ref[:, 3:4] * s
    gi = gx.astype(jnp.int32)
    gj = gy.astype(jnp.int32)
    lin = ((b * 3 + a) * ny + gj) * nx + gi
    return dict(a=a, valid=valid, b=b, gx=gx, gy=gy, gi=gi, gj=gj,
                tw=tw, th=th, lin=lin, c=t_ref[:, 1:2].astype(jnp.int32))


def _combine_body(shapes, t_ref, tT_ref, o0_ref, o1_ref, o2_ref,
                  ps0_ref, ps1_ref, ps2_ref, s0a_ref, img_ref,
                  total_ref, comps_ref):
    img = img_ref[0, 0]
    ps_refs = (ps0_ref, ps1_ref, ps2_ref)
    obj_refs = (o0_ref, o1_ref, o2_ref)
    lxy = jnp.float32(0.0)
    lwh = jnp.float32(0.0)
    lcls = jnp.float32(0.0)
    lobj = jnp.float32(0.0)
    for level, (ny, nx) in enumerate(shapes):
        col = _assign_cols(t_ref, img, level, ny, nx)
        s = jnp.float32(ny) / img

        twr = tT_ref[4:5, :]
        thr = tT_ref[5:6, :]
        iousr = []
        for k in range(3):
            aw = jnp.float32(_ANCHORS[level, k, 0])
            ah = jnp.float32(_ANCHORS[level, k, 1])
            inter = jnp.minimum(aw, twr) * jnp.minimum(ah, thr)
            iousr.append(inter / (aw * ah + twr * thr - inter))
        ar = jnp.where(iousr[1] > iousr[0], 1, 0)
        bestr = jnp.maximum(iousr[0], iousr[1])
        ar = jnp.where(iousr[2] > bestr, 2, ar)
        bestr = jnp.maximum(bestr, iousr[2])
        validr = bestr > _ANCHOR_T
        br = tT_ref[0:1, :].astype(jnp.int32)
        gir = (tT_ref[2:3, :] * s).astype(jnp.int32)
        gjr = (tT_ref[3:4, :] * s).astype(jnp.int32)
        linr = ((br * 3 + ar) * ny + gjr) * nx + gir

        w = col['valid'].astype(jnp.float32)
        cnt_valid = jnp.sum(w)
        safe = jnp.maximum(cnt_valid, 1.0)

        ps = ps_refs[level][...]

        tx = col['gx'] - col['gi'].astype(jnp.float32)
        ty = col['gy'] - col['gj'].astype(jnp.float32)
        px = jax.nn.sigmoid(ps[:, 0:1])
        py = jax.nn.sigmoid(ps[:, 1:2])
        lxy = lxy + jnp.sum(w * ((px - tx) ** 2 + (py - ty) ** 2)) / (safe * 2.0)

        aw_s = jnp.float32(_ANCHORS[level, 0, 0])
        ah_s = jnp.float32(_ANCHORS[level, 0, 1])
        for k in (1, 2):
            aw_s = jnp.where(col['a'] == k, _ANCHORS[level, k, 0], aw_s)
            ah_s = jnp.where(col['a'] == k, _ANCHORS[level, k, 1], ah_s)
        aw_s = aw_s * s
        ah_s = ah_s * s
        twx = jnp.log(jnp.maximum(col['tw'], 1e-9) / aw_s)
        twy = jnp.log(jnp.maximum(col['th'], 1e-9) / ah_s)
        lwh = lwh + jnp.sum(
            w * ((ps[:, 2:3] - twx) ** 2 + (ps[:, 3:4] - twy) ** 2)
        ) / (safe * 2.0)

        lane = jax.lax.broadcasted_iota(jnp.int32, ps.shape, 1)
        sp = _softplus(ps)
        sp_cls = jnp.sum(jnp.where(lane >= 5, sp, 0.0), axis=1, keepdims=True)
        x_true = jnp.sum(jnp.where(lane == 4 + col['c'], ps, 0.0),
                         axis=1, keepdims=True)
        lcls = lcls + jnp.sum(w * (sp_cls - x_true)) / (safe * 80.0)

        ssum = jnp.sum(_softplus(obj_refs[level][...]))
        if level == 0:
            ssum = ssum + s0a_ref[0, 0]
        eq = (col['lin'] == linr) & validr
        cnt = jnp.sum(eq.astype(jnp.float32), axis=1, keepdims=True)
        cnt = jnp.maximum(cnt, 1.0)
        x4 = jnp.sum(jnp.where(lane == 4, ps, 0.0), axis=1, keepdims=True)
        corr = jnp.sum(w * (x4 / cnt))
        ncells = jnp.float32(16 * 3 * ny * nx)
        lobj = lobj + (ssum - corr) / ncells

    lxy = lxy * _L_XY
    lwh = lwh * _L_WH
    lobj = lobj * _L_OBJ
    lcls = lcls * _L_CLS
    total_ref[...] = jnp.full((1, 1), (lxy + lwh + lobj + lcls) * 16.0,
                              dtype=jnp.float32)
    lane4 = jax.lax.broadcasted_iota(jnp.int32, (1, 4), 1)
    comps_ref[...] = (jnp.where(lane4 == 0, lxy, 0.0)
                      + jnp.where(lane4 == 1, lwh, 0.0)
                      + jnp.where(lane4 == 2, lobj, 0.0)
                      + jnp.where(lane4 == 3, lcls, 0.0))


def _combine_call(shapes, targets, targets_t, o0, o1, o2, ps0, ps1, ps2,
                  s0a, img):
    return pl.pallas_call(
        functools.partial(_combine_body, shapes),
        out_shape=(jax.ShapeDtypeStruct((1, 1), jnp.float32),
                   jax.ShapeDtypeStruct((1, 4), jnp.float32)),
    )(targets, targets_t, o0, o1, o2, ps0, ps1, ps2, s0a, img)


def kernel(p0, p1, p2, targets, image_size):
    shapes = [(p.shape[2], p.shape[3]) for p in (p0, p1, p2)]
    q0 = p0.reshape(-1, 85)
    q1 = p1.reshape(-1, 85)
    q2 = jnp.transpose(p2, (1, 2, 3, 0, 4)).reshape(-1, 85)
    imgf = jnp.float32(image_size)
    img = imgf.reshape(1, 1)

    tpad = jnp.pad(targets, ((0, _NT - targets.shape[0]), (0, 0)))
    tcols = [tpad[:, i] for i in range(6)]
    scales = jnp.repeat(
        jnp.array([s[0] for s in shapes], jnp.float32) / imgf, 16)

    o0, o1, o2, ps0, ps1, ps2 = _sc_stage(q0, q1, q2, tcols, scales)
    s0a = _tc_stream_call(q0)

    total, comps = _combine_call(
        shapes, tpad, tpad.T,
        o0.reshape(-1, 128), o1.reshape(-1, 128), o2.reshape(-1, 128),
        ps0, ps1, ps2, s0a, img)
    return (total.reshape(1), comps.reshape(4))


# speedup vs baseline: 1.0616x; 1.0616x over previous
import functools

import jax
import jax.numpy as jnp
import numpy as np
from jax import lax
from jax.experimental import pallas as pl
from jax.experimental.pallas import tpu as pltpu
from jax.experimental.pallas import tpu_sc as plsc

_ANCHORS = np.array([[[10.0, 13.0], [16.0, 30.0], [33.0, 23.0]],
                     [[30.0, 61.0], [62.0, 45.0], [59.0, 119.0]],
                     [[116.0, 90.0], [156.0, 198.0], [373.0, 326.0]]],
                    dtype=np.float32)
_ANCHOR_T = 0.2
_L_XY, _L_WH, _L_OBJ, _L_CLS = 2.0, 2.0, 1.0, 0.5

_NT = 1024
_NW = 32
_TPW = _NT // _NW
_CH = 240
_F = 99840
_RTC = 1920


def _softplus(x):
    return jnp.maximum(x, 0.0) + jnp.log1p(jnp.exp(-jnp.abs(x)))


def _sc_body(grids, q0_hbm, q1_hbm, q2_hbm, tcat_hbm,
             o0_hbm, o1_hbm, o2_hbm, p0_hbm, p1_hbm, p2_hbm,
             sbufA, sbufB, cbuf, tb0, tb2, tb3, tb4, tb5, scv, idxb, rowsb,
             semI, semO, semG):
    wid = lax.axis_index("s") * 2 + lax.axis_index("c")
    iota16 = lax.iota(jnp.int32, 16)
    col4 = jnp.full((16,), 4, jnp.int32)

    tbase = wid * _TPW
    pltpu.sync_copy(tcat_hbm.at[pl.ds(0 * _NT + tbase, _TPW)], tb0)
    pltpu.sync_copy(tcat_hbm.at[pl.ds(2 * _NT + tbase, _TPW)], tb2)
    pltpu.sync_copy(tcat_hbm.at[pl.ds(3 * _NT + tbase, _TPW)], tb3)
    pltpu.sync_copy(tcat_hbm.at[pl.ds(4 * _NT + tbase, _TPW)], tb4)
    pltpu.sync_copy(tcat_hbm.at[pl.ds(5 * _NT + tbase, _TPW)], tb5)
    pltpu.sync_copy(tcat_hbm.at[pl.ds(6 * _NT, 48)], scv)

    gdescs = []
    for level, (qh, (ny, nx)) in enumerate(
            ((q0_hbm, grids[0]), (q1_hbm, grids[1]), (q2_hbm, grids[2]))):
        for j in range(_TPW // 16):
            sl = pl.ds(j * 16, 16)
            s = scv[pl.ds(level * 16, 16)]
            tw = tb4[sl]
            th = tb5[sl]
            best = None
            a = None
            for k in range(3):
                aw = float(_ANCHORS[level, k, 0])
                ah = float(_ANCHORS[level, k, 1])
                inter = jnp.minimum(aw, tw) * jnp.minimum(ah, th)
                iou = inter / (aw * ah + tw * th - inter)
                if k == 0:
                    best = iou
                    a = jnp.zeros((16,), jnp.int32)
                else:
                    a = jnp.where(iou > best, k, a)
                    best = jnp.maximum(iou, best)
            b = tb0[sl].astype(jnp.int32)
            gi = (tb2[sl] * s).astype(jnp.int32)
            gj = (tb3[sl] * s).astype(jnp.int32)
            if level == 2:
                lin = ((a * ny + gj) * nx + gi) * 16 + b
            else:
                lin = ((b * 3 + a) * ny + gj) * nx + gi
            idxb[sl] = lin
        for j2 in range(_TPW // 16):
            linv = idxb[pl.ds(j2 * 16, 16)]
            for jj in range(16):
                j = j2 * 16 + jj
                r = linv[jj]
                gdescs.append(pltpu.async_copy(
                    qh.at[pl.ds(r, 1), :],
                    rowsb.at[pl.ds(level * _TPW + j, 1), :], semG))

    def extract(sbuf, cslot, nrows, clamp, row0, oh):
        ngroups = (nrows + 15) // 16

        def g_body(g, carry):
            rows = g * 16 + iota16
            if clamp:
                rows = jnp.minimum(rows, nrows - 1)
            vals = plsc.load_gather(sbuf, [rows, col4])
            cbuf[pl.ds(cslot * _CH + g * 16, 16)] = vals
            return carry

        lax.fori_loop(0, ngroups, g_body, 0)
        return pltpu.async_copy(cbuf.at[pl.ds(cslot * _CH, nrows)],
                                oh.at[pl.ds(row0, nrows)], semO)

    def stream_level(qh, oh, rows_pt, qoff):
        base = wid * rows_pt
        npairs = rows_pt // (2 * _CH)
        rem = rows_pt - 2 * npairs * _CH

        def in_copy(c, sbuf):
            return pltpu.async_copy(
                qh.at[pl.ds(qoff + base + c * _CH, _CH), :], sbuf, semI)

        in_copy(0, sbufA)

        def pair(pr, carry):
            c0 = 2 * pr
            in_copy(c0 + 1, sbufB)
            pltpu.make_async_copy(qh.at[pl.ds(qoff, _CH), :], sbufA,
                                  semI).wait()
            dA = extract(sbufA, 0, _CH, False, base + c0 * _CH, oh)

            @pl.when(pr < npairs - 1)
            def _():
                in_copy(c0 + 2, sbufA)

            pltpu.make_async_copy(qh.at[pl.ds(qoff, _CH), :], sbufB,
                                  semI).wait()
            dB = extract(sbufB, 1, _CH, False, base + (c0 + 1) * _CH, oh)
            dA.wait()
            dB.wait()
            return carry

        lax.fori_loop(0, npairs, pair, 0)
        if rem:
            row0 = base + 2 * npairs * _CH
            pltpu.sync_copy(qh.at[pl.ds(qoff + row0, rem), :],
                            sbufA.at[pl.ds(0, rem), :])
            extract(sbufA, 0, rem, True, row0, oh).wait()

    stream_level(q0_hbm, o0_hbm, (q0_hbm.shape[0] - _F) // _NW, _F)
    stream_level(q1_hbm, o1_hbm, q1_hbm.shape[0] // _NW, 0)
    stream_level(q2_hbm, o2_hbm, q2_hbm.shape[0] // _NW, 0)

    for d in gdescs:
        d.wait()
    for level, ph in enumerate((p0_hbm, p1_hbm, p2_hbm)):
        pltpu.sync_copy(rowsb.at[pl.ds(level * _TPW, _TPW), :],
                        ph.at[pl.ds(tbase, _TPW), :])


def _sc_stage(q0, q1, q2, tcat):
    grids = ((80, 80), (40, 40), (20, 20))
    n0, n1, n2 = q0.shape[0], q1.shape[0], q2.shape[0]
    mesh = plsc.VectorSubcoreMesh(core_axis_name="c", subcore_axis_name="s")
    kfn = functools.partial(
        pl.kernel,
        mesh=mesh,
        out_type=(jax.ShapeDtypeStruct((n0 - _F,), jnp.float32),
                  jax.ShapeDtypeStruct((n1,), jnp.float32),
                  jax.ShapeDtypeStruct((n2,), jnp.float32),
                  jax.ShapeDtypeStruct((_NT, 85), jnp.float32),
                  jax.ShapeDtypeStruct((_NT, 85), jnp.float32),
                  jax.ShapeDtypeStruct((_NT, 85), jnp.float32)),
        scratch_types=[
            pltpu.VMEM((_CH, 85), jnp.float32),
            pltpu.VMEM((_CH, 85), jnp.float32),
            pltpu.VMEM((2 * _CH,), jnp.float32),
            pltpu.VMEM((_TPW,), jnp.float32),
            pltpu.VMEM((_TPW,), jnp.float32),
            pltpu.VMEM((_TPW,), jnp.float32),
            pltpu.VMEM((_TPW,), jnp.float32),
            pltpu.VMEM((_TPW,), jnp.float32),
            pltpu.VMEM((48,), jnp.float32),
            pltpu.VMEM((_TPW,), jnp.int32),
            pltpu.VMEM((3 * _TPW, 85), jnp.float32),
            pltpu.SemaphoreType.DMA,
            pltpu.SemaphoreType.DMA,
            pltpu.SemaphoreType.DMA,
        ],
        compiler_params=pltpu.CompilerParams(needs_layout_passes=False),
    )(functools.partial(_sc_body, grids))
    return kfn(q0, q1, q2, tcat)


def _tc_stream_body(q_ref, out_ref):
    i = pl.program_id(0)

    @pl.when(i == 0)
    def _():
        out_ref[...] = jnp.zeros_like(out_ref)

    lane = jax.lax.broadcasted_iota(jnp.int32, (_RTC, 85), 1)
    x = jnp.where(lane == 4, q_ref[...], -1e9)
    out_ref[...] += jnp.full((1, 1), jnp.sum(jnp.log1p(jnp.exp(x))),
                             jnp.float32)


def _tc_stream_call(q0):
    return pl.pallas_call(
        _tc_stream_body,
        grid=(_F // _RTC,),
        in_specs=[pl.BlockSpec((_RTC, 85), lambda i: (i, 0))],
        out_specs=pl.BlockSpec((1, 1), lambda i: (0, 0)),
        out_shape=jax.ShapeDtypeStruct((1, 1), jnp.float32),
        compiler_params=pltpu.CompilerParams(
            dimension_semantics=("arbitrary",)),
    )(q0)


def _assign_cols(t_ref, img, level, ny, nx):
    s = jnp.float32(ny) / img
    tw = t_ref[:, 4:5] * s
    th = t_ref[:, 5:6] * s
    ious = []
    for k in range(3):
        aw = _ANCHORS[level, k, 0] * s
        ah = _ANCHORS[level, k, 1] * s
        inter = jnp.minimum(aw, tw) * jnp.minimum(ah, th)
        ious.append(inter / (aw * ah + tw * th - inter))
    a = jnp.where(ious[1] > ious[0], 1, 0)
    best = jnp.maximum(ious[0], ious[1])
    a = jnp.where(ious[2] > best, 2, a)
    best = jnp.maximum(best, ious[2])
    valid = best > _ANCHOR_T
    b = t_ref[:, 0:1].astype(jnp.int32)
    gx = t_ref[:, 2:3] * s
    gy = t_ref[:, 3:4] * s
    gi = gx.astype(jnp.int32)
    gj = gy.astype(jnp.int32)
    lin = ((b * 3 + a) * ny + gj) * nx + gi
    return dict(a=a, valid=valid, b=b, gx=gx, gy=gy, gi=gi, gj=gj,
                tw=tw, th=th, lin=lin, c=t_ref[:, 1:2].astype(jnp.int32))


def _combine_body(shapes, t_ref, tT_ref, o0_ref, o1_ref, o2_ref,
                  ps0_ref, ps1_ref, ps2_ref, s0a_ref, img_ref,
                  total_ref, comps_ref):
    img = img_ref[0, 0]
    ps_refs = (ps0_ref, ps1_ref, ps2_ref)
    obj_refs = (o0_ref, o1_ref, o2_ref)
    lxy = jnp.float32(0.0)
    lwh = jnp.float32(0.0)
    lcls = jnp.float32(0.0)
    lobj = jnp.float32(0.0)
    for level, (ny, nx) in enumerate(shapes):
        col = _assign_cols(t_ref, img, level, ny, nx)
        s = jnp.float32(ny) / img

        twr = tT_ref[4:5, :]
        thr = tT_ref[5:6, :]
        iousr = []
        for k in range(3):
            aw = jnp.float32(_ANCHORS[level, k, 0])
            ah = jnp.float32(_ANCHORS[level, k, 1])
            inter = jnp.minimum(aw, twr) * jnp.minimum(ah, thr)
            iousr.append(inter / (aw * ah + twr * thr - inter))
        ar = jnp.where(iousr[1] > iousr[0], 1, 0)
        bestr = jnp.maximum(iousr[0], iousr[1])
        ar = jnp.where(iousr[2] > bestr, 2, ar)
        bestr = jnp.maximum(bestr, iousr[2])
        validr = bestr > _ANCHOR_T
        br = tT_ref[0:1, :].astype(jnp.int32)
        gir = (tT_ref[2:3, :] * s).astype(jnp.int32)
        gjr = (tT_ref[3:4, :] * s).astype(jnp.int32)
        linr = ((br * 3 + ar) * ny + gjr) * nx + gir

        w = col['valid'].astype(jnp.float32)
        cnt_valid = jnp.sum(w)
        safe = jnp.maximum(cnt_valid, 1.0)

        ps = ps_refs[level][...]

        tx = col['gx'] - col['gi'].astype(jnp.float32)
        ty = col['gy'] - col['gj'].astype(jnp.float32)
        px = jax.nn.sigmoid(ps[:, 0:1])
        py = jax.nn.sigmoid(ps[:, 1:2])
        lxy = lxy + jnp.sum(w * ((px - tx) ** 2 + (py - ty) ** 2)) / (safe * 2.0)

        aw_s = jnp.float32(_ANCHORS[level, 0, 0])
        ah_s = jnp.float32(_ANCHORS[level, 0, 1])
        for k in (1, 2):
            aw_s = jnp.where(col['a'] == k, _ANCHORS[level, k, 0], aw_s)
            ah_s = jnp.where(col['a'] == k, _ANCHORS[level, k, 1], ah_s)
        aw_s = aw_s * s
        ah_s = ah_s * s
        twx = jnp.log(jnp.maximum(col['tw'], 1e-9) / aw_s)
        twy = jnp.log(jnp.maximum(col['th'], 1e-9) / ah_s)
        lwh = lwh + jnp.sum(
            w * ((ps[:, 2:3] - twx) ** 2 + (ps[:, 3:4] - twy) ** 2)
        ) / (safe * 2.0)

        lane = jax.lax.broadcasted_iota(jnp.int32, ps.shape, 1)
        sp = _softplus(ps)
        sp_cls = jnp.sum(jnp.where(lane >= 5, sp, 0.0), axis=1, keepdims=True)
        x_true = jnp.sum(jnp.where(lane == 4 + col['c'], ps, 0.0),
                         axis=1, keepdims=True)
        lcls = lcls + jnp.sum(w * (sp_cls - x_true)) / (safe * 80.0)

        ssum = jnp.sum(_softplus(obj_refs[level][...]))
        if level == 0:
            ssum = ssum + s0a_ref[0, 0]
        eq = (col['lin'] == linr) & validr
        cnt = jnp.sum(eq.astype(jnp.float32), axis=1, keepdims=True)
        cnt = jnp.maximum(cnt, 1.0)
        x4 = jnp.sum(jnp.where(lane == 4, ps, 0.0), axis=1, keepdims=True)
        corr = jnp.sum(w * (x4 / cnt))
        ncells = jnp.float32(16 * 3 * ny * nx)
        lobj = lobj + (ssum - corr) / ncells

    lxy = lxy * _L_XY
    lwh = lwh * _L_WH
    lobj = lobj * _L_OBJ
    lcls = lcls * _L_CLS
    total_ref[...] = jnp.full((1, 1), (lxy + lwh + lobj + lcls) * 16.0,
                              dtype=jnp.float32)
    lane4 = jax.lax.broadcasted_iota(jnp.int32, (1, 4), 1)
    comps_ref[...] = (jnp.where(lane4 == 0, lxy, 0.0)
                      + jnp.where(lane4 == 1, lwh, 0.0)
                      + jnp.where(lane4 == 2, lobj, 0.0)
                      + jnp.where(lane4 == 3, lcls, 0.0))


def _combine_call(shapes, targets, targets_t, o0, o1, o2, ps0, ps1, ps2,
                  s0a, img):
    return pl.pallas_call(
        functools.partial(_combine_body, shapes),
        out_shape=(jax.ShapeDtypeStruct((1, 1), jnp.float32),
                   jax.ShapeDtypeStruct((1, 4), jnp.float32)),
    )(targets, targets_t, o0, o1, o2, ps0, ps1, ps2, s0a, img)


def kernel(p0, p1, p2, targets, image_size):
    shapes = [(p.shape[2], p.shape[3]) for p in (p0, p1, p2)]
    q0 = p0.reshape(-1, 85)
    q1 = p1.reshape(-1, 85)
    q2 = jnp.transpose(p2, (1, 2, 3, 0, 4)).reshape(-1, 85)
    imgf = jnp.float32(image_size)
    img = imgf.reshape(1, 1)

    tpad = jnp.pad(targets, ((0, _NT - targets.shape[0]), (0, 0)))
    tpad_t = tpad.T
    scales = jnp.repeat(
        jnp.array([s[0] for s in shapes], jnp.float32) / imgf, 16)
    tcat = jnp.concatenate([tpad_t.reshape(-1), scales])

    o0, o1, o2, ps0, ps1, ps2 = _sc_stage(q0, q1, q2, tcat)
    s0a = _tc_stream_call(q0)

    total, comps = _combine_call(
        shapes, tpad, tpad_t,
        o0.reshape(-1, 128), o1.reshape(-1, 128), o2.reshape(-1, 128),
        ps0, ps1, ps2, s0a, img)
    return (total.reshape(1), comps.reshape(4))


# speedup vs baseline: 1.0619x; 1.0003x over previous
import functools

import jax
import jax.numpy as jnp
import numpy as np
from jax import lax
from jax.experimental import pallas as pl
from jax.experimental.pallas import tpu as pltpu
from jax.experimental.pallas import tpu_sc as plsc

_ANCHORS = np.array([[[10.0, 13.0], [16.0, 30.0], [33.0, 23.0]],
                     [[30.0, 61.0], [62.0, 45.0], [59.0, 119.0]],
                     [[116.0, 90.0], [156.0, 198.0], [373.0, 326.0]]],
                    dtype=np.float32)
_ANCHOR_T = 0.2
_L_XY, _L_WH, _L_OBJ, _L_CLS = 2.0, 2.0, 1.0, 0.5

_NT = 1024
_NW = 32
_TPW = _NT // _NW
_CH = 240
_F = 99840
_RTC = 1920


def _softplus(x):
    return jnp.maximum(x, 0.0) + jnp.log1p(jnp.exp(-jnp.abs(x)))


def _sc_body(grids, q0_hbm, q1_hbm, q2_hbm, tcat_hbm,
             o0_hbm, o1_hbm, o2_hbm, p0_hbm, p1_hbm, p2_hbm,
             sbufA, sbufB, cbuf, tb0, tb2, tb3, tb4, tb5, scv, idxb, rowsb,
             semI, semO, semG):
    wid = lax.axis_index("s") * 2 + lax.axis_index("c")
    iota16 = lax.iota(jnp.int32, 16)
    col4 = jnp.full((16,), 4, jnp.int32)

    tbase = wid * _TPW
    pltpu.sync_copy(tcat_hbm.at[pl.ds(0 * _NT + tbase, _TPW)], tb0)
    pltpu.sync_copy(tcat_hbm.at[pl.ds(2 * _NT + tbase, _TPW)], tb2)
    pltpu.sync_copy(tcat_hbm.at[pl.ds(3 * _NT + tbase, _TPW)], tb3)
    pltpu.sync_copy(tcat_hbm.at[pl.ds(4 * _NT + tbase, _TPW)], tb4)
    pltpu.sync_copy(tcat_hbm.at[pl.ds(5 * _NT + tbase, _TPW)], tb5)
    pltpu.sync_copy(tcat_hbm.at[pl.ds(6 * _NT, 48)], scv)

    gdescs = []
    for level, (qh, (ny, nx)) in enumerate(
            ((q0_hbm, grids[0]), (q1_hbm, grids[1]), (q2_hbm, grids[2]))):
        for j in range(_TPW // 16):
            sl = pl.ds(j * 16, 16)
            s = scv[pl.ds(level * 16, 16)]
            tw = tb4[sl]
            th = tb5[sl]
            a = jnp.zeros((16,), jnp.int32)
            bi = None
            bu = None
            for k in range(3):
                aw = float(_ANCHORS[level, k, 0])
                ah = float(_ANCHORS[level, k, 1])
                inter = jnp.minimum(aw, tw) * jnp.minimum(ah, th)
                union = aw * ah + tw * th - inter
                if k == 0:
                    bi, bu = inter, union
                else:
                    better = inter * bu > bi * union
                    a = jnp.where(better, k, a)
                    bi = jnp.where(better, inter, bi)
                    bu = jnp.where(better, union, bu)
            b = tb0[sl].astype(jnp.int32)
            gi = (tb2[sl] * s).astype(jnp.int32)
            gj = (tb3[sl] * s).astype(jnp.int32)
            if level == 2:
                lin = ((a * ny + gj) * nx + gi) * 16 + b
            else:
                lin = ((b * 3 + a) * ny + gj) * nx + gi
            idxb[sl] = lin
        for j2 in range(_TPW // 16):
            linv = idxb[pl.ds(j2 * 16, 16)]
            for jj in range(16):
                j = j2 * 16 + jj
                r = linv[jj]
                gdescs.append(pltpu.async_copy(
                    qh.at[pl.ds(r, 1), :],
                    rowsb.at[pl.ds(level * _TPW + j, 1), :], semG))

    def extract(sbuf, cslot, nrows, clamp, row0, oh):
        ngroups = (nrows + 15) // 16

        def g_body(g, carry):
            rows = g * 16 + iota16
            if clamp:
                rows = jnp.minimum(rows, nrows - 1)
            vals = plsc.load_gather(sbuf, [rows, col4])
            cbuf[pl.ds(cslot * _CH + g * 16, 16)] = vals
            return carry

        lax.fori_loop(0, ngroups, g_body, 0)
        return pltpu.async_copy(cbuf.at[pl.ds(cslot * _CH, nrows)],
                                oh.at[pl.ds(row0, nrows)], semO)

    def stream_level(qh, oh, rows_pt, qoff):
        base = wid * rows_pt
        npairs = rows_pt // (2 * _CH)
        rem = rows_pt - 2 * npairs * _CH

        def in_copy(c, sbuf):
            return pltpu.async_copy(
                qh.at[pl.ds(qoff + base + c * _CH, _CH), :], sbuf, semI)

        in_copy(0, sbufA)

        def pair(pr, carry):
            c0 = 2 * pr
            in_copy(c0 + 1, sbufB)
            pltpu.make_async_copy(qh.at[pl.ds(qoff, _CH), :], sbufA,
                                  semI).wait()
            dA = extract(sbufA, 0, _CH, False, base + c0 * _CH, oh)

            @pl.when(pr < npairs - 1)
            def _():
                in_copy(c0 + 2, sbufA)

            pltpu.make_async_copy(qh.at[pl.ds(qoff, _CH), :], sbufB,
                                  semI).wait()
            dB = extract(sbufB, 1, _CH, False, base + (c0 + 1) * _CH, oh)
            dA.wait()
            dB.wait()
            return carry

        lax.fori_loop(0, npairs, pair, 0)
        if rem:
            row0 = base + 2 * npairs * _CH
            pltpu.sync_copy(qh.at[pl.ds(qoff + row0, rem), :],
                            sbufA.at[pl.ds(0, rem), :])
            extract(sbufA, 0, rem, True, row0, oh).wait()

    stream_level(q0_hbm, o0_hbm, (q0_hbm.shape[0] - _F) // _NW, _F)
    stream_level(q1_hbm, o1_hbm, q1_hbm.shape[0] // _NW, 0)
    stream_level(q2_hbm, o2_hbm, q2_hbm.shape[0] // _NW, 0)

    for d in gdescs:
        d.wait()
    for level, ph in enumerate((p0_hbm, p1_hbm, p2_hbm)):
        pltpu.sync_copy(rowsb.at[pl.ds(level * _TPW, _TPW), :],
                        ph.at[pl.ds(tbase, _TPW), :])


def _sc_stage(q0, q1, q2, tcat):
    grids = ((80, 80), (40, 40), (20, 20))
    n0, n1, n2 = q0.shape[0], q1.shape[0], q2.shape[0]
    mesh = plsc.VectorSubcoreMesh(core_axis_name="c", subcore_axis_name="s")
    kfn = functools.partial(
        pl.kernel,
        mesh=mesh,
        out_type=(jax.ShapeDtypeStruct((n0 - _F,), jnp.float32),
                  jax.ShapeDtypeStruct((n1,), jnp.float32),
                  jax.ShapeDtypeStruct((n2,), jnp.float32),
                  jax.ShapeDtypeStruct((_NT, 85), jnp.float32),
                  jax.ShapeDtypeStruct((_NT, 85), jnp.float32),
                  jax.ShapeDtypeStruct((_NT, 85), jnp.float32)),
        scratch_types=[
            pltpu.VMEM((_CH, 85), jnp.float32),
            pltpu.VMEM((_CH, 85), jnp.float32),
            pltpu.VMEM((2 * _CH,), jnp.float32),
            pltpu.VMEM((_TPW,), jnp.float32),
            pltpu.VMEM((_TPW,), jnp.float32),
            pltpu.VMEM((_TPW,), jnp.float32),
            pltpu.VMEM((_TPW,), jnp.float32),
            pltpu.VMEM((_TPW,), jnp.float32),
            pltpu.VMEM((48,), jnp.float32),
            pltpu.VMEM((_TPW,), jnp.int32),
            pltpu.VMEM((3 * _TPW, 85), jnp.float32),
            pltpu.SemaphoreType.DMA,
            pltpu.SemaphoreType.DMA,
            pltpu.SemaphoreType.DMA,
        ],
        compiler_params=pltpu.CompilerParams(needs_layout_passes=False),
    )(functools.partial(_sc_body, grids))
    return kfn(q0, q1, q2, tcat)


def _tc_stream_body(q_ref, out_ref):
    i = pl.program_id(0)

    @pl.when(i == 0)
    def _():
        out_ref[...] = jnp.zeros_like(out_ref)

    lane = jax.lax.broadcasted_iota(jnp.int32, (_RTC, 85), 1)
    x = jnp.where(lane == 4, q_ref[...], -1e9)
    out_ref[...] += jnp.full((1, 1), jnp.sum(jnp.log1p(jnp.exp(x))),
                             jnp.float32)


def _tc_stream_call(q0):
    return pl.pallas_call(
        _tc_stream_body,
        grid=(_F // _RTC,),
        in_specs=[pl.BlockSpec((_RTC, 85), lambda i: (i, 0))],
        out_specs=pl.BlockSpec((1, 1), lambda i: (0, 0)),
        out_shape=jax.ShapeDtypeStruct((1, 1), jnp.float32),
        compiler_params=pltpu.CompilerParams(
            dimension_semantics=("arbitrary",)),
    )(q0)


def _assign_cols(t_ref, img, level, ny, nx):
    s = jnp.float32(ny) / img
    tw = t_ref[:, 4:5]
    th = t_ref[:, 5:6]
    a = jnp.zeros(tw.shape, jnp.int32)
    bi = None
    bu = None
    for k in range(3):
        aw = jnp.float32(_ANCHORS[level, k, 0])
        ah = jnp.float32(_ANCHORS[level, k, 1])
        inter = jnp.minimum(aw, tw) * jnp.minimum(ah, th)
        union = aw * ah + tw * th - inter
        if k == 0:
            bi, bu = inter, union
        else:
            better = inter * bu > bi * union
            a = jnp.where(better, k, a)
            bi = jnp.where(better, inter, bi)
            bu = jnp.where(better, union, bu)
    valid = bi > _ANCHOR_T * bu
    tw = tw * s
    th = th * s
    b = t_ref[:, 0:1].astype(jnp.int32)
    gx = t_ref[:, 2:3] * s
    gy = t_ref[:, 3:4] * s
    gi = gx.astype(jnp.int32)
    gj = gy.astype(jnp.int32)
    lin = ((b * 3 + a) * ny + gj) * nx + gi
    return dict(a=a, valid=valid, b=b, gx=gx, gy=gy, gi=gi, gj=gj,
                tw=tw, th=th, lin=lin, c=t_ref[:, 1:2].astype(jnp.int32))


def _combine_body(shapes, t_ref, tT_ref, o0_ref, o1_ref, o2_ref,
                  ps0_ref, ps1_ref, ps2_ref, s0a_ref, img_ref,
                  total_ref, comps_ref):
    img = img_ref[0, 0]
    ps_refs = (ps0_ref, ps1_ref, ps2_ref)
    obj_refs = (o0_ref, o1_ref, o2_ref)
    lxy = jnp.float32(0.0)
    lwh = jnp.float32(0.0)
    lcls = jnp.float32(0.0)
    lobj = jnp.float32(0.0)
    for level, (ny, nx) in enumerate(shapes):
        col = _assign_cols(t_ref, img, level, ny, nx)
        s = jnp.float32(ny) / img

        twr = tT_ref[4:5, :]
        thr = tT_ref[5:6, :]
        ar = jnp.zeros(twr.shape, jnp.int32)
        bir = None
        bur = None
        for k in range(3):
            aw = jnp.float32(_ANCHORS[level, k, 0])
            ah = jnp.float32(_ANCHORS[level, k, 1])
            inter = jnp.minimum(aw, twr) * jnp.minimum(ah, thr)
            union = aw * ah + twr * thr - inter
            if k == 0:
                bir, bur = inter, union
            else:
                better = inter * bur > bir * union
                ar = jnp.where(better, k, ar)
                bir = jnp.where(better, inter, bir)
                bur = jnp.where(better, union, bur)
        validr = bir > _ANCHOR_T * bur
        br = tT_ref[0:1, :].astype(jnp.int32)
        gir = (tT_ref[2:3, :] * s).astype(jnp.int32)
        gjr = (tT_ref[3:4, :] * s).astype(jnp.int32)
        linr = ((br * 3 + ar) * ny + gjr) * nx + gir

        w = col['valid'].astype(jnp.float32)
        cnt_valid = jnp.sum(w)
        safe = jnp.maximum(cnt_valid, 1.0)

        ps = ps_refs[level][...]

        tx = col['gx'] - col['gi'].astype(jnp.float32)
        ty = col['gy'] - col['gj'].astype(jnp.float32)
        px = jax.nn.sigmoid(ps[:, 0:1])
        py = jax.nn.sigmoid(ps[:, 1:2])
        lxy = lxy + jnp.sum(w * ((px - tx) ** 2 + (py - ty) ** 2)) / (safe * 2.0)

        aw_s = jnp.float32(_ANCHORS[level, 0, 0])
        ah_s = jnp.float32(_ANCHORS[level, 0, 1])
        for k in (1, 2):
            aw_s = jnp.where(col['a'] == k, _ANCHORS[level, k, 0], aw_s)
            ah_s = jnp.where(col['a'] == k, _ANCHORS[level, k, 1], ah_s)
        aw_s = aw_s * s
        ah_s = ah_s * s
        twx = jnp.log(jnp.maximum(col['tw'], 1e-9) / aw_s)
        twy = jnp.log(jnp.maximum(col['th'], 1e-9) / ah_s)
        lwh = lwh + jnp.sum(
            w * ((ps[:, 2:3] - twx) ** 2 + (ps[:, 3:4] - twy) ** 2)
        ) / (safe * 2.0)

        lane = jax.lax.broadcasted_iota(jnp.int32, ps.shape, 1)
        sp = _softplus(ps)
        sp_cls = jnp.sum(jnp.where(lane >= 5, sp, 0.0), axis=1, keepdims=True)
        x_true = jnp.sum(jnp.where(lane == 4 + col['c'], ps, 0.0),
                         axis=1, keepdims=True)
        lcls = lcls + jnp.sum(w * (sp_cls - x_true)) / (safe * 80.0)

        ssum = jnp.sum(_softplus(obj_refs[level][...]))
        if level == 0:
            ssum = ssum + s0a_ref[0, 0]
        eq = (col['lin'] == linr) & validr
        cnt = jnp.sum(eq.astype(jnp.float32), axis=1, keepdims=True)
        cnt = jnp.maximum(cnt, 1.0)
        x4 = jnp.sum(jnp.where(lane == 4, ps, 0.0), axis=1, keepdims=True)
        corr = jnp.sum(w * (x4 / cnt))
        ncells = jnp.float32(16 * 3 * ny * nx)
        lobj = lobj + (ssum - corr) / ncells

    lxy = lxy * _L_XY
    lwh = lwh * _L_WH
    lobj = lobj * _L_OBJ
    lcls = lcls * _L_CLS
    total_ref[...] = jnp.full((1, 1), (lxy + lwh + lobj + lcls) * 16.0,
                              dtype=jnp.float32)
    lane4 = jax.lax.broadcasted_iota(jnp.int32, (1, 4), 1)
    comps_ref[...] = (jnp.where(lane4 == 0, lxy, 0.0)
                      + jnp.where(lane4 == 1, lwh, 0.0)
                      + jnp.where(lane4 == 2, lobj, 0.0)
                      + jnp.where(lane4 == 3, lcls, 0.0))


def _combine_call(shapes, targets, targets_t, o0, o1, o2, ps0, ps1, ps2,
                  s0a, img):
    return pl.pallas_call(
        functools.partial(_combine_body, shapes),
        out_shape=(jax.ShapeDtypeStruct((1, 1), jnp.float32),
                   jax.ShapeDtypeStruct((1, 4), jnp.float32)),
    )(targets, targets_t, o0, o1, o2, ps0, ps1, ps2, s0a, img)


def kernel(p0, p1, p2, targets, image_size):
    shapes = [(p.shape[2], p.shape[3]) for p in (p0, p1, p2)]
    q0 = p0.reshape(-1, 85)
    q1 = p1.reshape(-1, 85)
    q2 = jnp.transpose(p2, (1, 2, 3, 0, 4)).reshape(-1, 85)
    imgf = jnp.float32(image_size)
    img = imgf.reshape(1, 1)

    tpad = jnp.pad(targets, ((0, _NT - targets.shape[0]), (0, 0)))
    tpad_t = tpad.T
    scales = jnp.repeat(
        jnp.array([s[0] for s in shapes], jnp.float32) / imgf, 16)
    tcat = jnp.concatenate([tpad_t.reshape(-1), scales])

    o0, o1, o2, ps0, ps1, ps2 = _sc_stage(q0, q1, q2, tcat)
    s0a = _tc_stream_call(q0)

    total, comps = _combine_call(
        shapes, tpad, tpad_t,
        o0.reshape(-1, 128), o1.reshape(-1, 128), o2.reshape(-1, 128),
        ps0, ps1, ps2, s0a, img)
    return (total.reshape(1), comps.reshape(4))
